# Initial kernel scaffold; baseline (speedup 1.0000x reference)
import functools

import jax
import jax.numpy as jnp
from jax import lax
from jax.experimental import pallas as pl
from jax.experimental.pallas import tpu as pltpu
from jax.experimental.pallas import tpu_sc as plsc

N_LAYERS = 3
NC = 2
NS = 16
NW = NC * NS
C = 500
K = 128


def _off_at(obuf, i):
    g = (i // 16) * 16
    v = obuf[pl.ds(g, 16)]
    m = lax.iota(jnp.int32, 16) == (i - g)
    return jnp.sum(jnp.where(m, v, 0))


def _make_spmm(n, d, e_pad, nchunk, off_pad):
    ch_per_w = (nchunk + NW - 1) // NW
    mesh = plsc.VectorSubcoreMesh(core_axis_name="c", subcore_axis_name="s")

    @functools.partial(
        pl.kernel,
        out_type=jax.ShapeDtypeStruct((n * d,), jnp.float32),
        mesh=mesh,
        scratch_types=[
            pltpu.VMEM((C * d,), jnp.float32),
            pltpu.VMEM((K, d), jnp.float32),
            pltpu.VMEM((K,), jnp.int32),
            pltpu.VMEM((K,), jnp.float32),
            pltpu.VMEM((K,), jnp.int32),
            pltpu.VMEM((off_pad,), jnp.int32),
            pltpu.SemaphoreType.DMA,
        ],
    )
    def spmm(x_hbm, cols_hbm, vals_hbm, rows_hbm, off_hbm, y_hbm,
             acc, gbuf, cbuf, vbuf, rbuf, obuf, sem):
        wid = lax.axis_index("s") * NC + lax.axis_index("c")
        pltpu.sync_copy(off_hbm, obuf)
        zero16 = jnp.zeros((16,), jnp.float32)

        def process_chunk(chunk):
            r0 = chunk * C
            e_lo = _off_at(obuf, chunk)
            e_hi = _off_at(obuf, chunk + 1)

            def zero_body(i, _):
                acc[pl.ds(i * 16, 16)] = zero16
                return 0
            lax.fori_loop(0, C * d // 16, zero_body, 0)

            e_al = (e_lo // 8) * 8
            nb = (e_hi - e_al + K - 1) // K

            def batch_body(b, _):
                base = e_al + b * K
                pltpu.sync_copy(cols_hbm.at[pl.ds(base, K)], cbuf)
                pltpu.sync_copy(vals_hbm.at[pl.ds(base, K)], vbuf)
                pltpu.sync_copy(rows_hbm.at[pl.ds(base, K)], rbuf)
                pltpu.async_copy(x_hbm.at[cbuf], gbuf, sem).wait()
                lo = jnp.maximum(e_lo - base, 0)
                hi = jnp.minimum(e_hi - base, K)

                def edge_body(e, _):
                    g = (e // 16) * 16
                    m = lax.iota(jnp.int32, 16) == (e - g)
                    row = jnp.sum(jnp.where(m, rbuf[pl.ds(g, 16)], 0))
                    val = jnp.sum(jnp.where(m, vbuf[pl.ds(g, 16)], 0.0))
                    rbase = (row - r0) * d
                    for db in range(d // 16):
                        sl = pl.ds(rbase + db * 16, 16)
                        acc[sl] = acc[sl] + val * gbuf[e, pl.ds(db * 16, 16)]
                    return 0
                lax.fori_loop(lo, hi, edge_body, 0)
                return 0
            lax.fori_loop(0, nb, batch_body, 0)
            pltpu.sync_copy(acc, y_hbm.at[pl.ds(r0 * d, C * d)])

        for t in range(ch_per_w):
            chunk = wid + t * NW
            if (t + 1) * NW <= nchunk:
                process_chunk(chunk)
            else:
                @pl.when(chunk < nchunk)
                def _():
                    process_chunk(chunk)

    return spmm


def _norm_acc_kernel(w, y_ref, res_ref, x_ref, out_ref):
    y = y_ref[...]
    ss = jnp.sum(y * y, axis=1, keepdims=True)
    inv = lax.rsqrt(jnp.maximum(ss, 1e-24))
    x = y * inv
    x_ref[...] = x
    out_ref[...] = res_ref[...] + x * w


def _make_norm(n, d, w):
    br = 400
    grid = n // br
    return pl.pallas_call(
        functools.partial(_norm_acc_kernel, w),
        grid=(grid,),
        in_specs=[
            pl.BlockSpec((br, d), lambda i: (i, 0)),
            pl.BlockSpec((br, d), lambda i: (i, 0)),
        ],
        out_specs=[
            pl.BlockSpec((br, d), lambda i: (i, 0)),
            pl.BlockSpec((br, d), lambda i: (i, 0)),
        ],
        out_shape=[
            jax.ShapeDtypeStruct((n, d), jnp.float32),
            jax.ShapeDtypeStruct((n, d), jnp.float32),
        ],
    )


def kernel(in_embs, edge_vals, edge_rows, edge_cols):
    n, d = in_embs.shape
    e = edge_rows.shape[0]
    assert n % C == 0
    nchunk = n // C
    off_pad = ((nchunk + 1 + 15) // 16) * 16
    e_pad = (e // K + 2) * K

    boundaries = (jnp.arange(nchunk + 1, dtype=jnp.int32) * C)
    off = jnp.searchsorted(edge_rows, boundaries, side="left").astype(jnp.int32)
    off = jnp.pad(off, (0, off_pad - (nchunk + 1)))
    cols_p = jnp.pad(edge_cols, (0, e_pad - e))
    vals_p = jnp.pad(edge_vals, (0, e_pad - e))
    rows_p = jnp.pad(edge_rows, (0, e_pad - e))

    spmm = _make_spmm(n, d, e_pad, nchunk, off_pad)

    res = in_embs
    x = in_embs
    for i in range(N_LAYERS):
        y = spmm(x, cols_p, vals_p, rows_p, off).reshape(n, d)
        x, res = _make_norm(n, d, 1.0 / (i + 1))(y, res)
    return res


# speedup vs baseline: 2.6322x; 2.6322x over previous
import functools

import jax
import jax.numpy as jnp
from jax import lax
from jax.experimental import pallas as pl
from jax.experimental.pallas import tpu as pltpu
from jax.experimental.pallas import tpu_sc as plsc

N_LAYERS = 3
NC = 2
NS = 16
NW = NC * NS
C = 500
K = 128


def _scal(ref, i):
    return ref[pl.ds(i, 16)][0]


def _make_spmm(n, d, e_pad, nchunk, off_pad):
    ch_per_w = (nchunk + NW - 1) // NW
    mesh = plsc.VectorSubcoreMesh(core_axis_name="c", subcore_axis_name="s")

    @functools.partial(
        pl.kernel,
        out_type=jax.ShapeDtypeStruct((n * d,), jnp.float32),
        mesh=mesh,
        scratch_types=[
            pltpu.VMEM((C * d,), jnp.float32),
            pltpu.VMEM((K, d), jnp.float32),
            pltpu.VMEM((K,), jnp.int32),
            pltpu.VMEM((K + 16,), jnp.float32),
            pltpu.VMEM((K + 16,), jnp.int32),
            pltpu.VMEM((off_pad + 16,), jnp.int32),
            pltpu.SemaphoreType.DMA,
        ],
    )
    def spmm(x_hbm, cols_hbm, vals_hbm, rows_hbm, off_hbm, y_hbm,
             acc, gbuf, cbuf, vbuf, rbuf, obuf, sem):
        wid = lax.axis_index("s") * NC + lax.axis_index("c")
        pltpu.sync_copy(off_hbm, obuf.at[pl.ds(0, off_pad)])
        zero16 = jnp.zeros((16,), jnp.float32)

        def process_chunk(chunk):
            r0 = chunk * C
            e_lo = _scal(obuf, chunk)
            e_hi = _scal(obuf, chunk + 1)

            def zero_body(i, _):
                acc[pl.ds(i * 16, 16)] = zero16
                return 0
            lax.fori_loop(0, C * d // 16, zero_body, 0)

            e_al = (e_lo // 8) * 8
            nb = (e_hi - e_al + K - 1) // K

            def batch_body(b, _):
                base = e_al + b * K
                pltpu.sync_copy(cols_hbm.at[pl.ds(base, K)], cbuf)
                pltpu.sync_copy(vals_hbm.at[pl.ds(base, K)], vbuf.at[pl.ds(0, K)])
                pltpu.sync_copy(rows_hbm.at[pl.ds(base, K)], rbuf.at[pl.ds(0, K)])
                pltpu.async_copy(x_hbm.at[cbuf], gbuf, sem).wait()
                lo = jnp.maximum(e_lo - base, 0)
                hi = jnp.minimum(e_hi - base, K)

                def edge_body(e, _):
                    row = _scal(rbuf, e)
                    val = _scal(vbuf, e)
                    rbase = (row - r0) * d
                    for db in range(d // 16):
                        sl = pl.ds(rbase + db * 16, 16)
                        acc[sl] = acc[sl] + val * gbuf[e, pl.ds(db * 16, 16)]
                    return 0
                lax.fori_loop(lo, hi, edge_body, 0)
                return 0
            lax.fori_loop(0, nb, batch_body, 0)
            pltpu.sync_copy(acc, y_hbm.at[pl.ds(r0 * d, C * d)])

        for t in range(ch_per_w):
            chunk = wid + t * NW
            if (t + 1) * NW <= nchunk:
                process_chunk(chunk)
            else:
                @pl.when(chunk < nchunk)
                def _():
                    process_chunk(chunk)

    return spmm


def _norm_acc_kernel(w, y_ref, res_ref, x_ref, out_ref):
    y = y_ref[...]
    ss = jnp.sum(y * y, axis=1, keepdims=True)
    inv = lax.rsqrt(jnp.maximum(ss, 1e-24))
    x = y * inv
    x_ref[...] = x
    out_ref[...] = res_ref[...] + x * w


def _make_norm(n, d, w):
    br = 400
    grid = n // br
    return pl.pallas_call(
        functools.partial(_norm_acc_kernel, w),
        grid=(grid,),
        in_specs=[
            pl.BlockSpec((br, d), lambda i: (i, 0)),
            pl.BlockSpec((br, d), lambda i: (i, 0)),
        ],
        out_specs=[
            pl.BlockSpec((br, d), lambda i: (i, 0)),
            pl.BlockSpec((br, d), lambda i: (i, 0)),
        ],
        out_shape=[
            jax.ShapeDtypeStruct((n, d), jnp.float32),
            jax.ShapeDtypeStruct((n, d), jnp.float32),
        ],
    )


def kernel(in_embs, edge_vals, edge_rows, edge_cols):
    n, d = in_embs.shape
    e = edge_rows.shape[0]
    assert n % C == 0
    nchunk = n // C
    off_pad = ((nchunk + 1 + 15) // 16) * 16
    e_pad = (e // K + 2) * K

    boundaries = (jnp.arange(nchunk + 1, dtype=jnp.int32) * C)
    off = jnp.searchsorted(edge_rows, boundaries, side="left").astype(jnp.int32)
    off = jnp.pad(off, (0, off_pad - (nchunk + 1)))
    cols_p = jnp.pad(edge_cols, (0, e_pad - e))
    vals_p = jnp.pad(edge_vals, (0, e_pad - e))
    rows_p = jnp.pad(edge_rows, (0, e_pad - e))

    spmm = _make_spmm(n, d, e_pad, nchunk, off_pad)

    res = in_embs
    x = in_embs
    for i in range(N_LAYERS):
        y = spmm(x, cols_p, vals_p, rows_p, off).reshape(n, d)
        x, res = _make_norm(n, d, 1.0 / (i + 1))(y, res)
    return res
